# Initial kernel scaffold; baseline (speedup 1.0000x reference)
import jax
import jax.numpy as jnp
from jax.experimental import pallas as pl


def kernel(hough_map):
    raise NotImplementedError("write your pallas kernel here")


# speedup vs baseline: 5.6033x; 5.6033x over previous
import functools

import jax
import jax.numpy as jnp
import numpy as np
from jax import lax
from jax.experimental import pallas as pl
from jax.experimental.pallas import tpu as pltpu
from jax.experimental.pallas import tpu_sc as plsc

_H = 384
_W = 384
_A = 180
_R = 180
_AP = 184
_FLAGS = _AP * _AP
_NV = _FLAGS // 16
_NSUB = 16
_ROWS = _H // _NSUB
_STRIP = _ROWS * _W


def _nms_body(p_ref, f_ref):
    P = p_ref[...]
    xc = P[:, 1:185, 1:185]
    pooled = xc
    for di in range(3):
        for dj in range(3):
            pooled = jnp.maximum(pooled, P[:, di:di + 184, dj:dj + 184])
    gmax = jnp.max(P, axis=(1, 2), keepdims=True)
    flag = jnp.logical_and(xc == pooled, xc > 0.5 * gmax)
    f_ref[...] = flag.astype(jnp.float32)


_nms = pl.pallas_call(
    _nms_body,
    out_shape=jax.ShapeDtypeStruct((2, _AP, _AP), jnp.float32),
)


def _raster_body(flags_hbm, tab_hbm, out_hbm, flags_v, tab_v, strip_v, pkbuf):
    c = lax.axis_index("c")
    s = lax.axis_index("s")

    pltpu.sync_copy(flags_hbm.at[c], flags_v)
    pltpu.sync_copy(tab_hbm, tab_v)

    zeros16 = jnp.zeros((16,), jnp.float32)
    ones16 = jnp.ones((16,), jnp.float32)

    def _zero(i, carry):
        strip_v[pl.ds(i * 16, 16)] = zeros16
        return carry

    lax.fori_loop(0, _STRIP // 16, _zero, 0)

    iota = lax.iota(jnp.int32, 16)
    row0 = s * _ROWS
    ok1 = iota < 8
    base0 = iota * _W
    base1 = jnp.where(ok1, (iota + 16) * _W, 0)

    def _gather(idx):
        return plsc.load_gather(tab_v.at[:], [idx])

    y0 = _gather(row0 + iota + 576)
    y1 = _gather(jnp.minimum(row0 + 16 + iota, _H - 1) + 576)

    def _process_peak(k):
        qv = plsc.load_gather(pkbuf.at[:], [jnp.full((16,), k, jnp.int32)])
        av = (qv * 45591) >> 23
        rv = qv - av * _AP
        cs = _gather(av)
        sn = _gather(av + 192)
        rh = _gather(rv + 384)
        inv = 1.0 / cs
        hw = 3.0 * jnp.abs(inv)
        marg = 2.5 + 2e-4 * hw

        def _group(yv, basev, okv):
            uc = (rh - sn * yv) * inv + 191.5
            jlo = jnp.clip(uc - hw - marg, 0.0, 384.0)
            jhi = jnp.clip(uc + hw + marg, 0.0, 384.0)
            jstart = jnp.maximum(jlo.astype(jnp.int32) - 1, 0)
            mcnt = jhi.astype(jnp.int32) - jstart + 2
            if okv is not None:
                mcnt = jnp.where(okv, mcnt, 0)
            m = jnp.max(mcnt)

            def _col(kk, carry):
                j = jstart + kk
                jsafe = jnp.where(j < _W, j, 0)
                x = _gather(jsafe + 576)
                d = cs * x + sn * yv - rh
                pred = (jnp.abs(d) < 3.0) & (j < _W)
                if okv is not None:
                    pred = pred & okv
                plsc.store_scatter(strip_v.at[:], [basev + jsafe],
                                   ones16, mask=pred)
                return carry

            lax.fori_loop(0, m, _col, 0)

        _group(y0, base0, None)
        _group(y1, base1, ok1)

    def _scan(i, carry):
        fv = flags_v[pl.ds(i * 16, 16)]
        msk = fv > 0.0
        cnt = jnp.sum(jnp.where(msk, 1, 0))

        @pl.when(cnt > 0)
        def _have():
            plsc.store_compressed(pkbuf.at[:], i * 16 + iota, mask=msk)

            def _pk(k, carry2):
                _process_peak(k)
                return carry2

            lax.fori_loop(0, cnt, _pk, 0)

        return carry

    lax.fori_loop(0, _NV, _scan, 0)

    pltpu.sync_copy(strip_v, out_hbm.at[c, pl.ds(row0 * _W, _STRIP)])


_raster = functools.partial(
    pl.kernel,
    out_type=jax.ShapeDtypeStruct((2, _H * _W), jnp.float32),
    mesh=plsc.VectorSubcoreMesh(core_axis_name="c", subcore_axis_name="s",
                                num_cores=2, num_subcores=_NSUB),
    compiler_params=pltpu.CompilerParams(needs_layout_passes=False),
    scratch_types=[
        pltpu.VMEM((_FLAGS,), jnp.float32),
        pltpu.VMEM((960,), jnp.float32),
        pltpu.VMEM((_STRIP,), jnp.float32),
        pltpu.VMEM((16,), jnp.int32),
    ],
)(_raster_body)


def kernel(hough_map):
    h = hough_map[:, 0]
    P = jnp.full((2, 192, 192), -jnp.inf, dtype=jnp.float32)
    P = P.at[:, 1:181, 1:181].set(h)
    flags = _nms(P).reshape(2, _FLAGS)

    hz = hough_map.reshape(-1)[0] * 0.0
    theta = jnp.arange(_A, dtype=jnp.float32) * (np.pi / _A) + hz
    bf = jnp.bfloat16
    probe = jnp.zeros((2, 128), jnp.float32)
    probe = probe.at[0, 0].set(1.0).at[1, 1].set(1.0)

    def _tab_body(carry, theta_a):
        cos_a = jnp.cos(theta_a)
        sin_a = jnp.sin(theta_a)
        normal_vec = jnp.stack([cos_a, sin_a], axis=0)
        rho_cal = normal_vec @ probe
        return carry, rho_cal[:2]

    _, cs_pairs = lax.scan(_tab_body, 0, theta)
    cos_t = cs_pairs[:, 0]
    sin_t = cs_pairs[:, 1]
    max_rho = jnp.sqrt(jnp.asarray((_W / 2.0) ** 2 + (_H / 2.0) ** 2,
                                   dtype=jnp.float32))
    delta_rho = 2.0 * max_rho / (_R - 1)
    rho_all = (jnp.arange(_R, dtype=jnp.float32) - _R / 2.0) * delta_rho
    coord = (jnp.arange(_W, dtype=jnp.float32)
             - (_W - 1) / 2.0).astype(bf).astype(jnp.float32)
    pad = jnp.zeros((12,), jnp.float32)
    tab = jnp.concatenate([cos_t, jnp.ones((12,), jnp.float32),
                           sin_t, pad, rho_all, pad, coord])

    out = _raster(flags, tab)
    return out.reshape(2, 1, _H, _W)


# speedup vs baseline: 6.5579x; 1.1704x over previous
import functools

import jax
import jax.numpy as jnp
import numpy as np
from jax import lax
from jax.experimental import pallas as pl
from jax.experimental.pallas import tpu as pltpu
from jax.experimental.pallas import tpu_sc as plsc

_H = 384
_W = 384
_A = 180
_R = 180
_AP = 184
_FLAGS = _AP * _AP
_NV = _FLAGS // 16
_NSUB = 16
_ROWS = _H // _NSUB
_STRIP = _ROWS * _W


def _nms_body(p_ref, f_ref):
    P = p_ref[...]
    xc = P[:, 1:185, 1:185]
    pooled = xc
    for di in range(3):
        for dj in range(3):
            pooled = jnp.maximum(pooled, P[:, di:di + 184, dj:dj + 184])
    gmax = jnp.max(P, axis=(1, 2), keepdims=True)
    flag = jnp.logical_and(xc == pooled, xc > 0.5 * gmax)
    f_ref[...] = flag.astype(jnp.float32)


_nms = pl.pallas_call(
    _nms_body,
    out_shape=jax.ShapeDtypeStruct((2, _AP, _AP), jnp.float32),
)


def _raster_body(flags_hbm, tab_hbm, out_hbm, flags_v, tab_v, strip_v):
    c = lax.axis_index("c")
    s = lax.axis_index("s")

    pltpu.sync_copy(flags_hbm.at[c], flags_v)
    pltpu.sync_copy(tab_hbm, tab_v)

    zeros16 = jnp.zeros((16,), jnp.float32)
    ones16 = jnp.ones((16,), jnp.float32)

    def _zero(i, carry):
        strip_v[pl.ds(i * 16, 16)] = zeros16
        return carry

    lax.fori_loop(0, _STRIP // 16 + 1, _zero, 0)

    iota = lax.iota(jnp.int32, 16)
    row0 = s * _ROWS
    ok1 = iota < 8
    base0 = iota * _W
    base1 = jnp.where(ok1, (iota + 16) * _W, 0)

    def _gather(idx):
        return plsc.load_gather(tab_v.at[:], [idx])

    y0 = _gather(row0 + iota + 576)
    y1 = _gather(jnp.minimum(row0 + 16 + iota, _H - 1) + 576)

    def _process_peak(qv):
        av = ((qv.astype(jnp.float32) + 0.5)
              * jnp.float32(1.0 / _AP)).astype(jnp.int32)
        rv = qv - av * _AP
        cs = _gather(av)
        sn = _gather(av + 192)
        rh = _gather(rv + 384)
        inv = 1.0 / cs
        hw = 3.0 * jnp.abs(inv)
        marg = 2.5 + 2e-4 * hw

        def _group(yv, basev, okv):
            uc = (rh - sn * yv) * inv + 191.5
            jlo = jnp.clip(uc - hw - marg, 0.0, 384.0)
            jhi = jnp.clip(uc + hw + marg, 0.0, 384.0)
            jstart = jnp.maximum(jlo.astype(jnp.int32) - 1, 0)
            mcnt = jhi.astype(jnp.int32) - jstart + 2
            if okv is not None:
                mcnt = jnp.where(okv, mcnt, 0)
            m = jnp.max(mcnt)

            def _col(kk, carry):
                j = jstart + kk
                jsafe = jnp.where(j < _W, j, 0)
                x = _gather(jsafe + 576)
                d = cs * x + sn * yv - rh
                pred = (jnp.abs(d) < 3.0) & (j < _W)
                if okv is not None:
                    pred = pred & okv
                idx = jnp.where(pred, basev + jsafe, _STRIP + iota)
                plsc.store_scatter(strip_v.at[:], [idx], ones16)
                return carry

            lax.fori_loop(0, m, _col, 0)

        _group(y0, base0, None)
        _group(y1, base1, ok1)

    def _scan(i, carry):
        fv = flags_v[pl.ds(i * 16, 16)]
        msk0 = fv > 0.0
        cnt = jnp.sum(jnp.where(msk0, 1, 0))

        @pl.when(cnt > 0)
        def _have():
            def _pk(k, mvec):
                msk = mvec > 0
                first = jnp.min(jnp.where(msk, iota, 16))
                qv = jnp.full((16,), i * 16, jnp.int32) + first
                _process_peak(qv)
                return jnp.where(iota == first, 0, mvec)

            lax.fori_loop(0, cnt, _pk, jnp.where(msk0, 1, 0))

        return carry

    lax.fori_loop(0, _NV, _scan, 0)

    pltpu.sync_copy(strip_v.at[pl.ds(0, _STRIP)],
                    out_hbm.at[c, pl.ds(row0 * _W, _STRIP)])


_raster = functools.partial(
    pl.kernel,
    out_type=jax.ShapeDtypeStruct((2, _H * _W), jnp.float32),
    mesh=plsc.VectorSubcoreMesh(core_axis_name="c", subcore_axis_name="s",
                                num_cores=2, num_subcores=_NSUB),
    compiler_params=pltpu.CompilerParams(needs_layout_passes=False),
    scratch_types=[
        pltpu.VMEM((_FLAGS,), jnp.float32),
        pltpu.VMEM((960,), jnp.float32),
        pltpu.VMEM((_STRIP + 16,), jnp.float32),
    ],
)(_raster_body)


def kernel(hough_map):
    h = hough_map[:, 0]
    P = jnp.full((2, 192, 192), -jnp.inf, dtype=jnp.float32)
    P = P.at[:, 1:181, 1:181].set(h)
    flags = _nms(P).reshape(2, _FLAGS)

    hz = hough_map.reshape(-1)[0] * 0.0
    theta = jnp.arange(_A, dtype=jnp.float32) * (np.pi / _A) + hz
    probe = jnp.zeros((2, 128), jnp.float32)
    probe = probe.at[0, 0].set(1.0).at[1, 1].set(1.0)

    def _tab_body(carry, theta_a):
        cos_a = jnp.cos(theta_a)
        sin_a = jnp.sin(theta_a)
        normal_vec = jnp.stack([cos_a, sin_a], axis=0)
        rho_cal = normal_vec @ probe
        return carry, rho_cal[:2]

    _, cs_pairs = lax.scan(_tab_body, 0, theta, unroll=12)
    cos_t = cs_pairs[:, 0]
    sin_t = cs_pairs[:, 1]
    max_rho = jnp.sqrt(jnp.asarray((_W / 2.0) ** 2 + (_H / 2.0) ** 2,
                                   dtype=jnp.float32))
    delta_rho = 2.0 * max_rho / (_R - 1)
    rho_all = (jnp.arange(_R, dtype=jnp.float32) - _R / 2.0) * delta_rho
    coordf = jnp.arange(_W, dtype=jnp.float32) - (_W - 1) / 2.0
    cb = lax.bitcast_convert_type(coordf, jnp.int32)
    cb = (cb + 0x7FFF + ((cb >> 16) & 1)) & ~0xFFFF
    coord = lax.bitcast_convert_type(cb, jnp.float32)
    pad = jnp.zeros((12,), jnp.float32)
    tab = jnp.concatenate([cos_t, jnp.ones((12,), jnp.float32),
                           sin_t, pad, rho_all, pad, coord])

    out = _raster(flags, tab)
    return out.reshape(2, 1, _H, _W)
